# Initial kernel scaffold; baseline (speedup 1.0000x reference)
import jax
import jax.numpy as jnp
from jax.experimental import pallas as pl


def kernel(x, noise):
    raise NotImplementedError("write your pallas kernel here")


# speedup vs baseline: 210.2701x; 210.2701x over previous
import jax
import jax.numpy as jnp
from jax.experimental import pallas as pl

_PMIN = 0.1
_ROWS_PER_BLOCK = 16


def _mask_scale_kernel(x_ref, noise_ref, out_ref):
    p = jnp.float32(_PMIN)
    inv = jnp.float32(1.0) / (jnp.float32(1.0) - p)
    x = x_ref[...]
    noise = noise_ref[...]
    out_ref[...] = jnp.where(noise > p, x * inv, jnp.float32(0.0))


def kernel(x, noise):
    m, n = x.shape
    grid = (m // _ROWS_PER_BLOCK,)
    spec = pl.BlockSpec((_ROWS_PER_BLOCK, n), lambda i: (i, 0))
    return pl.pallas_call(
        _mask_scale_kernel,
        grid=grid,
        in_specs=[spec, spec],
        out_specs=spec,
        out_shape=jax.ShapeDtypeStruct((m, n), jnp.float32),
    )(x, noise)


# speedup vs baseline: 217.5865x; 1.0348x over previous
import jax
import jax.numpy as jnp
from jax.experimental import pallas as pl

_PMIN = 0.1
_ROWS_PER_BLOCK = 32


def _mask_scale_kernel(x_ref, noise_ref, out_ref):
    p = jnp.float32(_PMIN)
    inv = jnp.float32(1.0) / (jnp.float32(1.0) - p)
    x = x_ref[...]
    noise = noise_ref[...]
    out_ref[...] = jnp.where(noise > p, x * inv, jnp.float32(0.0))


def kernel(x, noise):
    m, n = x.shape
    grid = (m // _ROWS_PER_BLOCK,)
    spec = pl.BlockSpec((_ROWS_PER_BLOCK, n), lambda i: (i, 0))
    return pl.pallas_call(
        _mask_scale_kernel,
        grid=grid,
        in_specs=[spec, spec],
        out_specs=spec,
        out_shape=jax.ShapeDtypeStruct((m, n), jnp.float32),
    )(x, noise)
